# baseline (speedup 1.0000x reference)
import jax
import jax.numpy as jnp
from jax.experimental import pallas as pl
from jax.experimental.pallas import tpu as pltpu

_BM = 512


def _matmul_body(x_ref, w_ref, b_ref, o_ref):
    x = x_ref[...].astype(jnp.bfloat16)
    o_ref[...] = (
        jnp.dot(x, w_ref[...], preferred_element_type=jnp.float32) + b_ref[...]
    )


def kernel(inputs, W, b):
    m, k = inputs.shape
    n = W.shape[1]
    w_bf16 = W.astype(jnp.bfloat16)
    b2 = b.reshape(1, n)
    grid = (m // _BM,)
    return pl.pallas_call(
        _matmul_body,
        grid=grid,
        in_specs=[
            pl.BlockSpec((_BM, k), lambda i: (i, 0)),
            pl.BlockSpec((k, n), lambda i: (0, 0)),
            pl.BlockSpec((1, n), lambda i: (0, 0)),
        ],
        out_specs=pl.BlockSpec((_BM, n), lambda i: (i, 0)),
        out_shape=jax.ShapeDtypeStruct((m, n), jnp.float32),
        compiler_params=pltpu.CompilerParams(
            dimension_semantics=("arbitrary",),
        ),
    )(inputs, w_bf16, b2)


# speedup vs baseline: 1.0420x; 1.0420x over previous
import jax
import jax.numpy as jnp
from jax.experimental import pallas as pl
from jax.experimental.pallas import tpu as pltpu

_BM = 512


def _matmul_body(x_ref, w_ref, b_ref, o_ref, w_bf16_ref):
    @pl.when(pl.program_id(0) == 0)
    def _cast_w():
        w_bf16_ref[...] = w_ref[...].astype(jnp.bfloat16)

    x = x_ref[...].astype(jnp.bfloat16)
    o_ref[...] = (
        jnp.dot(x, w_bf16_ref[...], preferred_element_type=jnp.float32)
        + b_ref[...]
    )


def kernel(inputs, W, b):
    m, k = inputs.shape
    n = W.shape[1]
    b2 = b.reshape(1, n)
    grid = (m // _BM,)
    return pl.pallas_call(
        _matmul_body,
        grid=grid,
        in_specs=[
            pl.BlockSpec((_BM, k), lambda i: (i, 0)),
            pl.BlockSpec((k, n), lambda i: (0, 0)),
            pl.BlockSpec((1, n), lambda i: (0, 0)),
        ],
        out_specs=pl.BlockSpec((_BM, n), lambda i: (i, 0)),
        out_shape=jax.ShapeDtypeStruct((m, n), jnp.float32),
        scratch_shapes=[pltpu.VMEM((k, n), jnp.bfloat16)],
        compiler_params=pltpu.CompilerParams(
            dimension_semantics=("arbitrary",),
        ),
    )(inputs, W, b2)
